# Initial kernel scaffold; baseline (speedup 1.0000x reference)
import jax
import jax.numpy as jnp
from jax.experimental import pallas as pl


def kernel(x, src_mask, mel_mask, max_mel_len, pitch_truth, energy_truth, duration_truth, dur_params, pitch_params, energy_params, pitch_emb, energy_emb, pitch_bins, energy_bins):
    raise NotImplementedError("write your pallas kernel here")


# speedup vs baseline: 16.9251x; 16.9251x over previous
import functools

import jax
import jax.numpy as jnp
from jax import lax
from jax.experimental import pallas as pl
from jax.experimental.pallas import tpu as pltpu
from jax.experimental.pallas import tpu_sc as plsc

B, S, D, F, K, NBINS, MAXMEL = 16, 512, 512, 512, 3, 256, 4096
TBL = S + 1


def _ln(h, g, be):
    mu = jnp.mean(h, axis=1, keepdims=True)
    c = h - mu
    v = jnp.mean(c * c, axis=1, keepdims=True)
    return c * lax.rsqrt(v + 1e-5) * g + be


def _mm3(v, wr):
    rows = lax.broadcasted_iota(jnp.int32, (S, 1), 0)
    vm1 = jnp.where(rows == 0, 0.0, pltpu.roll(v, 1, 0))
    vp1 = jnp.where(rows == S - 1, 0.0, pltpu.roll(v, S - 1, 0))
    y = jnp.dot(v, wr[1], preferred_element_type=jnp.float32)
    y = y + jnp.dot(vm1, wr[0], preferred_element_type=jnp.float32)
    y = y + jnp.dot(vp1, wr[2], preferred_element_type=jnp.float32)
    return y


def _predict(v, w1, b1, g1, be1, w2, b2, g2, be2, wl, bl):
    h = jnp.maximum(_mm3(v, w1) + b1[...], 0.0)
    h = _ln(h, g1[...], be1[...])
    h = jnp.maximum(_mm3(h, w2) + b2[...], 0.0)
    h = _ln(h, g2[...], be2[...])
    return jnp.sum(h * wl[...], axis=1, keepdims=True) + bl[0, 0]


def _bucket_add(truth_col, bins_row, emb):
    idx = jnp.sum((bins_row[...] < truth_col).astype(jnp.float32), axis=1,
                  keepdims=True)
    oh = (lax.broadcasted_iota(jnp.int32, (S, NBINS), 1).astype(jnp.float32)
          == idx)
    return jnp.dot(oh.astype(jnp.float32), emb[...],
                   preferred_element_type=jnp.float32)


def _tc_body(xr, ptr, etr, durr,
             pbins, ebins, pemb, eemb,
             pw1, pb1, pg1, pbe1, pw2, pb2, pg2, pbe2, pwl, pbl,
             ew1, eb1, eg1, ebe1, ew2, eb2, eg2, ebe2, ewl, ebl,
             dw1, db1, dg1, dbe1, dw2, db2, dg2, dbe2, dwl, dbl,
             xaugr, ppredr, epredr, dpredr, gidxr, mlenr):
    b = pl.program_id(0)
    x0 = xr[0]
    ppredr[0] = _predict(x0, pw1, pb1, pg1, pbe1, pw2, pb2, pg2, pbe2,
                         pwl, pbl)
    x1 = x0 + _bucket_add(ptr[0], pbins, pemb)
    epredr[0] = _predict(x1, ew1, eb1, eg1, ebe1, ew2, eb2, eg2, ebe2,
                         ewl, ebl)
    x2 = x1 + _bucket_add(etr[0], ebins, eemb)
    dpredr[0] = _predict(x2, dw1, db1, dg1, dbe1, dw2, db2, dg2, dbe2,
                         dwl, dbl)
    xaugr[0, :S, :] = x2
    xaugr[0, S:, :] = jnp.zeros((1, D), jnp.float32)

    dur_row = durr[0]
    r = lax.broadcasted_iota(jnp.int32, (S, S), 0)
    c = lax.broadcasted_iota(jnp.int32, (S, S), 1)
    tri = (r <= c).astype(jnp.float32)
    cum = jnp.dot(dur_row, tri, preferred_element_type=jnp.float32)
    total = cum[:, S - 1:S]
    trow = lax.broadcasted_iota(jnp.int32, (MAXMEL, 1), 0).astype(jnp.float32)
    idx = jnp.sum((cum <= trow).astype(jnp.float32), axis=1, keepdims=True)
    gidx = jnp.where(trow < total, idx, float(S)) + (b * TBL).astype(
        jnp.float32)
    gidxr[0] = gidx.astype(jnp.int32)
    mlenr[0] = total.astype(jnp.int32)


def _wspec(shape):
    n = len(shape)
    return pl.BlockSpec(shape, lambda b: (0,) * n)


def _tc_call(x, ptc, etc_, durr, pbins, ebins, pemb, eemb, wlist):
    in_specs = [
        pl.BlockSpec((1, S, D), lambda b: (b, 0, 0)),
        pl.BlockSpec((1, S, 1), lambda b: (b, 0, 0)),
        pl.BlockSpec((1, S, 1), lambda b: (b, 0, 0)),
        pl.BlockSpec((1, 1, S), lambda b: (b, 0, 0)),
        _wspec((1, NBINS)), _wspec((1, NBINS)),
        _wspec((NBINS, D)), _wspec((NBINS, D)),
    ]
    for _ in range(3):
        in_specs += [
            _wspec((K, D, F)), _wspec((1, F)), _wspec((1, F)), _wspec((1, F)),
            _wspec((K, F, F)), _wspec((1, F)), _wspec((1, F)), _wspec((1, F)),
            _wspec((1, F)), _wspec((1, 1)),
        ]
    out_specs = [
        pl.BlockSpec((1, TBL, D), lambda b: (b, 0, 0)),
        pl.BlockSpec((1, S, 1), lambda b: (b, 0, 0)),
        pl.BlockSpec((1, S, 1), lambda b: (b, 0, 0)),
        pl.BlockSpec((1, S, 1), lambda b: (b, 0, 0)),
        pl.BlockSpec((1, MAXMEL, 1), lambda b: (b, 0, 0)),
        pl.BlockSpec((1, 1, 1), lambda b: (b, 0, 0)),
    ]
    out_shapes = [
        jax.ShapeDtypeStruct((B, TBL, D), jnp.float32),
        jax.ShapeDtypeStruct((B, S, 1), jnp.float32),
        jax.ShapeDtypeStruct((B, S, 1), jnp.float32),
        jax.ShapeDtypeStruct((B, S, 1), jnp.float32),
        jax.ShapeDtypeStruct((B, MAXMEL, 1), jnp.int32),
        jax.ShapeDtypeStruct((B, 1, 1), jnp.int32),
    ]
    return pl.pallas_call(
        _tc_body,
        grid=(B,),
        in_specs=in_specs,
        out_specs=out_specs,
        out_shape=out_shapes,
        compiler_params=pltpu.CompilerParams(
            dimension_semantics=("arbitrary",),
            vmem_limit_bytes=110 * 1024 * 1024,
        ),
    )(x, ptc, etc_, durr, pbins, ebins, pemb, eemb, *wlist)


_ROWS_PER_W = B * MAXMEL // 32
_CH = 64
_NIT = _ROWS_PER_W // _CH


def _sc_body(table_hbm, gidx_hbm, out_hbm, idx_v, buf, sem):
    wid = lax.axis_index("s") * 2 + lax.axis_index("c")
    base = wid * _ROWS_PER_W

    def body(i, carry):
        off = base + i * _CH
        pltpu.sync_copy(gidx_hbm.at[pl.ds(off, _CH)], idx_v)
        pltpu.async_copy(table_hbm.at[idx_v], buf, sem).wait()
        pltpu.sync_copy(buf, out_hbm.at[pl.ds(off, _CH)])
        return carry

    lax.fori_loop(0, _NIT, body, 0)


@functools.partial(jax.jit, static_argnums=())
def _sc_call(table, gidx):
    return pl.kernel(
        _sc_body,
        mesh=plsc.VectorSubcoreMesh(core_axis_name="c", subcore_axis_name="s"),
        out_type=jax.ShapeDtypeStruct((B * MAXMEL, D), jnp.float32),
        scratch_types=[
            pltpu.VMEM((_CH,), jnp.int32),
            pltpu.VMEM((_CH, D), jnp.float32),
            pltpu.SemaphoreType.DMA,
        ],
    )(table, gidx)


def _prep_pred(p):
    return [
        jnp.transpose(p['W1'], (2, 1, 0)), p['b1'].reshape(1, F),
        p['g1'].reshape(1, F), p['be1'].reshape(1, F),
        jnp.transpose(p['W2'], (2, 1, 0)), p['b2'].reshape(1, F),
        p['g2'].reshape(1, F), p['be2'].reshape(1, F),
        p['Wl'].reshape(1, F), p['bl'].reshape(1, 1),
    ]


def kernel(x, src_mask, mel_mask, max_mel_len, pitch_truth, energy_truth,
           duration_truth, dur_params, pitch_params, energy_params,
           pitch_emb, energy_emb, pitch_bins, energy_bins):
    ptc = pitch_truth.reshape(B, S, 1)
    etc_ = energy_truth.reshape(B, S, 1)
    durr = duration_truth.astype(jnp.float32).reshape(B, 1, S)
    big = jnp.full((1, 1), 3.0e38, jnp.float32)
    pbins = jnp.concatenate([pitch_bins.reshape(1, NBINS - 1), big], axis=1)
    ebins = jnp.concatenate([energy_bins.reshape(1, NBINS - 1), big], axis=1)
    wlist = (_prep_pred(pitch_params) + _prep_pred(energy_params)
             + _prep_pred(dur_params))

    xaug, ppred, epred, dpred, gidx, mlen = _tc_call(
        x, ptc, etc_, durr, pbins, ebins, pitch_emb, energy_emb, wlist)

    x_out = _sc_call(xaug.reshape(B * TBL, D),
                     gidx.reshape(B * MAXMEL)).reshape(B, MAXMEL, D)

    zero = jnp.float32(0.0)
    pitch_pred = jnp.where(src_mask, zero, ppred.reshape(B, S))
    energy_pred = jnp.where(src_mask, zero, epred.reshape(B, S))
    log_dur_pred = jnp.where(src_mask, zero, dpred.reshape(B, S))
    mel_len = mlen.reshape(B)
    return (x_out, pitch_pred, energy_pred, log_dur_pred, mel_len, mel_mask)


# speedup vs baseline: 17.0750x; 1.0089x over previous
import functools

import jax
import jax.numpy as jnp
from jax import lax
from jax.experimental import pallas as pl
from jax.experimental.pallas import tpu as pltpu
from jax.experimental.pallas import tpu_sc as plsc

B, S, D, F, K, NBINS, MAXMEL = 16, 512, 512, 512, 3, 256, 4096
TBL = S + 1


def _ln(h, g, be):
    mu = jnp.mean(h, axis=1, keepdims=True)
    c = h - mu
    v = jnp.mean(c * c, axis=1, keepdims=True)
    return c * lax.rsqrt(v + 1e-5) * g + be


def _mm3(v, wr):
    rows = lax.broadcasted_iota(jnp.int32, (S, 1), 0)
    vm1 = jnp.where(rows == 0, 0.0, pltpu.roll(v, 1, 0))
    vp1 = jnp.where(rows == S - 1, 0.0, pltpu.roll(v, S - 1, 0))
    y = jnp.dot(v, wr[1], preferred_element_type=jnp.float32)
    y = y + jnp.dot(vm1, wr[0], preferred_element_type=jnp.float32)
    y = y + jnp.dot(vp1, wr[2], preferred_element_type=jnp.float32)
    return y


def _predict(v, w1, b1, g1, be1, w2, b2, g2, be2, wl, bl):
    h = jnp.maximum(_mm3(v, w1) + b1[...], 0.0)
    h = _ln(h, g1[...], be1[...])
    h = jnp.maximum(_mm3(h, w2) + b2[...], 0.0)
    h = _ln(h, g2[...], be2[...])
    return jnp.sum(h * wl[...], axis=1, keepdims=True) + bl[0, 0]


def _bucket_add(truth_col, bins_row, emb):
    idx = jnp.sum((bins_row[...] < truth_col).astype(jnp.float32), axis=1,
                  keepdims=True)
    oh = (lax.broadcasted_iota(jnp.int32, (S, NBINS), 1).astype(jnp.float32)
          == idx)
    return jnp.dot(oh.astype(jnp.float32), emb[...],
                   preferred_element_type=jnp.float32)


def _tc_body(xr, ptr, etr, durr,
             pbins, ebins, pemb, eemb,
             pw1, pb1, pg1, pbe1, pw2, pb2, pg2, pbe2, pwl, pbl,
             ew1, eb1, eg1, ebe1, ew2, eb2, eg2, ebe2, ewl, ebl,
             dw1, db1, dg1, dbe1, dw2, db2, dg2, dbe2, dwl, dbl,
             xaugr, ppredr, epredr, dpredr, gidxr, mlenr):
    b = pl.program_id(0)
    x0 = xr[0]
    ppredr[0] = _predict(x0, pw1, pb1, pg1, pbe1, pw2, pb2, pg2, pbe2,
                         pwl, pbl)
    x1 = x0 + _bucket_add(ptr[0], pbins, pemb)
    epredr[0] = _predict(x1, ew1, eb1, eg1, ebe1, ew2, eb2, eg2, ebe2,
                         ewl, ebl)
    x2 = x1 + _bucket_add(etr[0], ebins, eemb)
    dpredr[0] = _predict(x2, dw1, db1, dg1, dbe1, dw2, db2, dg2, dbe2,
                         dwl, dbl)
    xaugr[0, :S, :] = x2
    xaugr[0, S:, :] = jnp.zeros((1, D), jnp.float32)

    dur_row = durr[0]
    r = lax.broadcasted_iota(jnp.int32, (S, S), 0)
    c = lax.broadcasted_iota(jnp.int32, (S, S), 1)
    tri = (r <= c).astype(jnp.float32)
    cum = jnp.dot(dur_row, tri, preferred_element_type=jnp.float32)
    total = cum[:, S - 1:S]
    trow = lax.broadcasted_iota(jnp.int32, (MAXMEL, 1), 0).astype(jnp.float32)
    idx = jnp.sum((cum <= trow).astype(jnp.float32), axis=1, keepdims=True)
    gidx = jnp.where(trow < total, idx, float(S)) + (b * TBL).astype(
        jnp.float32)
    gidxr[0] = gidx.astype(jnp.int32)
    mlenr[0] = total.astype(jnp.int32)


def _wspec(shape):
    n = len(shape)
    return pl.BlockSpec(shape, lambda b: (0,) * n)


def _tc_call(x, ptc, etc_, durr, pbins, ebins, pemb, eemb, wlist):
    in_specs = [
        pl.BlockSpec((1, S, D), lambda b: (b, 0, 0)),
        pl.BlockSpec((1, S, 1), lambda b: (b, 0, 0)),
        pl.BlockSpec((1, S, 1), lambda b: (b, 0, 0)),
        pl.BlockSpec((1, 1, S), lambda b: (b, 0, 0)),
        _wspec((1, NBINS)), _wspec((1, NBINS)),
        _wspec((NBINS, D)), _wspec((NBINS, D)),
    ]
    for _ in range(3):
        in_specs += [
            _wspec((K, D, F)), _wspec((1, F)), _wspec((1, F)), _wspec((1, F)),
            _wspec((K, F, F)), _wspec((1, F)), _wspec((1, F)), _wspec((1, F)),
            _wspec((1, F)), _wspec((1, 1)),
        ]
    out_specs = [
        pl.BlockSpec((1, TBL, D), lambda b: (b, 0, 0)),
        pl.BlockSpec((1, S, 1), lambda b: (b, 0, 0)),
        pl.BlockSpec((1, S, 1), lambda b: (b, 0, 0)),
        pl.BlockSpec((1, S, 1), lambda b: (b, 0, 0)),
        pl.BlockSpec((1, MAXMEL, 1), lambda b: (b, 0, 0)),
        pl.BlockSpec((1, 1, 1), lambda b: (b, 0, 0)),
    ]
    out_shapes = [
        jax.ShapeDtypeStruct((B, TBL, D), jnp.float32),
        jax.ShapeDtypeStruct((B, S, 1), jnp.float32),
        jax.ShapeDtypeStruct((B, S, 1), jnp.float32),
        jax.ShapeDtypeStruct((B, S, 1), jnp.float32),
        jax.ShapeDtypeStruct((B, MAXMEL, 1), jnp.int32),
        jax.ShapeDtypeStruct((B, 1, 1), jnp.int32),
    ]
    return pl.pallas_call(
        _tc_body,
        grid=(B,),
        in_specs=in_specs,
        out_specs=out_specs,
        out_shape=out_shapes,
        compiler_params=pltpu.CompilerParams(
            dimension_semantics=("arbitrary",),
            vmem_limit_bytes=110 * 1024 * 1024,
        ),
    )(x, ptc, etc_, durr, pbins, ebins, pemb, eemb, *wlist)


_ROWS_PER_W = B * MAXMEL // 32
_CH = 64
_NIT = _ROWS_PER_W // _CH


_NBUF = 3


def _sc_body(table_hbm, gidx_hbm, out_hbm, idx_v, b0, b1, b2,
             g0, g1, g2, s0, s1, s2):
    wid = lax.axis_index("s") * 2 + lax.axis_index("c")
    base = wid * _ROWS_PER_W
    bufs, gsems, ssems = (b0, b1, b2), (g0, g1, g2), (s0, s1, s2)
    pltpu.sync_copy(gidx_hbm.at[pl.ds(base, _ROWS_PER_W)], idx_v)

    def gather(c, b):
        pltpu.async_copy(
            table_hbm.at[idx_v.at[pl.ds(c * _CH, _CH)]], bufs[b], gsems[b])

    for c in range(_NBUF):
        gather(c, c)
    for c in range(_NIT):
        b = c % _NBUF
        pltpu.make_async_copy(
            table_hbm.at[idx_v.at[pl.ds(0, _CH)]], bufs[b], gsems[b]).wait()
        dst = out_hbm.at[pl.ds(base + c * _CH, _CH)]
        pltpu.async_copy(bufs[b], dst, ssems[b])
        if c + _NBUF < _NIT:
            pltpu.make_async_copy(bufs[b], dst, ssems[b]).wait()
            gather(c + _NBUF, b)
        else:
            pltpu.make_async_copy(bufs[b], dst, ssems[b]).wait()


@functools.partial(jax.jit, static_argnums=())
def _sc_call(table, gidx):
    return pl.kernel(
        _sc_body,
        mesh=plsc.VectorSubcoreMesh(core_axis_name="c", subcore_axis_name="s"),
        out_type=jax.ShapeDtypeStruct((B * MAXMEL, D), jnp.float32),
        scratch_types=[
            pltpu.VMEM((_ROWS_PER_W,), jnp.int32),
            pltpu.VMEM((_CH, D), jnp.float32),
            pltpu.VMEM((_CH, D), jnp.float32),
            pltpu.VMEM((_CH, D), jnp.float32),
            pltpu.SemaphoreType.DMA, pltpu.SemaphoreType.DMA,
            pltpu.SemaphoreType.DMA, pltpu.SemaphoreType.DMA,
            pltpu.SemaphoreType.DMA, pltpu.SemaphoreType.DMA,
        ],
    )(table, gidx)


def _prep_pred(p):
    return [
        jnp.transpose(p['W1'], (2, 1, 0)), p['b1'].reshape(1, F),
        p['g1'].reshape(1, F), p['be1'].reshape(1, F),
        jnp.transpose(p['W2'], (2, 1, 0)), p['b2'].reshape(1, F),
        p['g2'].reshape(1, F), p['be2'].reshape(1, F),
        p['Wl'].reshape(1, F), p['bl'].reshape(1, 1),
    ]


def kernel(x, src_mask, mel_mask, max_mel_len, pitch_truth, energy_truth,
           duration_truth, dur_params, pitch_params, energy_params,
           pitch_emb, energy_emb, pitch_bins, energy_bins):
    ptc = pitch_truth.reshape(B, S, 1)
    etc_ = energy_truth.reshape(B, S, 1)
    durr = duration_truth.astype(jnp.float32).reshape(B, 1, S)
    big = jnp.full((1, 1), 3.0e38, jnp.float32)
    pbins = jnp.concatenate([pitch_bins.reshape(1, NBINS - 1), big], axis=1)
    ebins = jnp.concatenate([energy_bins.reshape(1, NBINS - 1), big], axis=1)
    wlist = (_prep_pred(pitch_params) + _prep_pred(energy_params)
             + _prep_pred(dur_params))

    xaug, ppred, epred, dpred, gidx, mlen = _tc_call(
        x, ptc, etc_, durr, pbins, ebins, pitch_emb, energy_emb, wlist)

    x_out = _sc_call(xaug.reshape(B * TBL, D),
                     gidx.reshape(B * MAXMEL)).reshape(B, MAXMEL, D)

    zero = jnp.float32(0.0)
    pitch_pred = jnp.where(src_mask, zero, ppred.reshape(B, S))
    energy_pred = jnp.where(src_mask, zero, epred.reshape(B, S))
    log_dur_pred = jnp.where(src_mask, zero, dpred.reshape(B, S))
    mel_len = mlen.reshape(B)
    return (x_out, pitch_pred, energy_pred, log_dur_pred, mel_len, mel_mask)
